# baseline (speedup 1.0000x reference)
import functools

import jax
import jax.numpy as jnp
from jax import lax
from jax.experimental import pallas as pl
from jax.experimental.pallas import tpu as pltpu
from jax.experimental.pallas import tpu_sc as plsc

F32 = jnp.float32
I32 = jnp.int32

N_NODES = 50000
N_COMM = 5000
E = 400000
N_MULTI = 10000

NW = 32
E_PAD = 401408
EPT = E_PAD // NW
GA_C = 256
GA_CHUNKS = EPT // GA_C

VT = 64
RPV = 784
N_NODES_ACC = VT * RPV
CK = 1568
CONV_CHUNKS = E_PAD // CK
SB = 64

CPT = 160
N_COMM_ACC = NW * CPT
NSCAN = 50176
PK = 1568
NCHUNK_P = NSCAN // PK
MSCAN = 10240
MK = 2048
MCHUNK_P = MSCAN // MK
SBP = 64

NODE_BLK = 2000
EDGE_BLK = 2000
COMM_BLK = 1000

@functools.lru_cache(maxsize=None)
def _sc_mesh():
    return plsc.VectorSubcoreMesh(core_axis_name="c", subcore_axis_name="s")


def _wid():
    return lax.axis_index("s") * 2 + lax.axis_index("c")


def _full(shape):
    nd = len(shape)
    return pl.BlockSpec(shape, lambda i: (0,) * nd)


def _embed_call(x, w1, b1, w2, b2, w3, b3):
    def body(x_ref, w1r, b1r, w2r, b2r, w3r, b3r, h_ref):
        xb = x_ref[...]
        x1 = jax.nn.relu(xb[:, :8] @ w1r[...] + b1r[...])
        x2 = jax.nn.relu(xb[:, 8:] @ w2r[...] + b2r[...])
        hh = jnp.concatenate([x1, x2], axis=1)
        h_ref[...] = jax.nn.relu(hh @ w3r[...] + b3r[...])

    return pl.pallas_call(
        body,
        grid=(N_NODES // NODE_BLK,),
        in_specs=[
            pl.BlockSpec((NODE_BLK, 20), lambda i: (i, 0)),
            _full((8, 64)), _full((1, 64)),
            _full((12, 64)), _full((1, 64)),
            _full((128, 128)), _full((1, 128)),
        ],
        out_specs=pl.BlockSpec((NODE_BLK, 128), lambda i: (i, 0)),
        out_shape=jax.ShapeDtypeStruct((N_NODES, 128), F32),
    )(x, w1, b1, w2, b2, w3, b3)


def _node_mats_call(h, wbs):
    k = len(wbs)

    def body(*refs):
        h_ref = refs[0]
        w_refs = refs[1:1 + k]
        b_refs = refs[1 + k:1 + 2 * k]
        out_refs = refs[1 + 2 * k:]
        hb = h_ref[...]
        for j in range(k):
            out_refs[j][...] = hb @ w_refs[j][...] + b_refs[j][...]

    out_specs = [pl.BlockSpec((NODE_BLK, 128), lambda i: (i, 0))] * k
    out_shape = [jax.ShapeDtypeStruct((N_NODES, 128), F32)] * k
    args = [h] + [w for (w, _) in wbs] + [b for (_, b) in wbs]
    return pl.pallas_call(
        body,
        grid=(N_NODES // NODE_BLK,),
        in_specs=[pl.BlockSpec((NODE_BLK, 128), lambda i: (i, 0))]
        + [_full((128, 128))] * k + [_full((1, 128))] * k,
        out_specs=out_specs,
        out_shape=out_shape,
    )(*args)


def _edge_g_call(z0, ea, wee, bee, wel, bel, wg1, bg1, wg2, bg2):
    def body(z_ref, ea_ref, weer, beer, welr, belr, wg1r, bg1r, wg2r, bg2r,
             o1, o2):
        e2 = jax.nn.relu(ea_ref[...] @ weer[...] + beer[...])
        e1 = e2 @ welr[...] + belr[...]
        mask = jnp.tanh(z_ref[...] + e1)
        o1[...] = jax.nn.sigmoid(mask @ wg1r[...] + bg1r[...])
        o2[...] = jax.nn.sigmoid(mask @ wg2r[...] + bg2r[...])

    eo = [pl.BlockSpec((EDGE_BLK, 128), lambda i: (i, 0))] * 2
    es = [jax.ShapeDtypeStruct((E, 128), F32)] * 2
    return pl.pallas_call(
        body,
        grid=(E // EDGE_BLK,),
        in_specs=[
            pl.BlockSpec((EDGE_BLK, 128), lambda i: (i, 0)),
            pl.BlockSpec((EDGE_BLK, 16), lambda i: (i, 0)),
            _full((16, 64)), _full((1, 64)),
            _full((64, 128)), _full((1, 128)),
            _full((128, 128)), _full((1, 128)),
            _full((128, 128)), _full((1, 128)),
        ],
        out_specs=eo,
        out_shape=es,
    )(z0, ea, wee, bee, wel, bel, wg1, bg1, wg2, bg2)


def _combine_call(ri, ai, ra, aa, wbs):
    k = len(wbs)

    def body(*refs):
        rir, air, rar, aar = refs[:4]
        w_refs = refs[4:4 + k]
        b_refs = refs[4 + k:4 + 2 * k]
        outs = refs[4 + 2 * k:]
        hb = jax.nn.relu(rir[...] + air[...]) + jax.nn.relu(rar[...] + aar[...])
        outs[0][...] = hb
        for j in range(k):
            outs[1 + j][...] = hb @ w_refs[j][...] + b_refs[j][...]

    out_specs = [pl.BlockSpec((NODE_BLK, 128), lambda i: (i, 0))] * (1 + k)
    out_shape = [jax.ShapeDtypeStruct((N_NODES, 128), F32)] * (1 + k)

    args = [ri, ai, ra, aa] + [w for (w, _) in wbs] + [b for (_, b) in wbs]
    return pl.pallas_call(
        body,
        grid=(N_NODES // NODE_BLK,),
        in_specs=[pl.BlockSpec((NODE_BLK, 128), lambda i: (i, 0))] * 4
        + [_full((128, 128))] * k + [_full((1, 128))] * k,
        out_specs=out_specs,
        out_shape=out_shape,
    )(*args)


def _final_call(s1, c1, m1, s2, c2, m2, wl1, bl1, wl2, bl2):
    def body(s1r, c1r, m1r, s2r, c2r, m2r, w1r, b1r, w2r, b2r, out_ref):
        mean1 = s1r[...] / jnp.maximum(c1r[...][:, :1], 1.0)
        mean2 = s2r[...] / jnp.maximum(c2r[...][:, :1], 1.0)
        g = (jnp.concatenate([mean1, m1r[...]], axis=1)
             + jnp.concatenate([mean2, m2r[...]], axis=1))
        gg = jax.nn.relu(g @ w1r[...] + b1r[...])
        out_ref[...] = gg @ w2r[...] + b2r[...]

    return pl.pallas_call(
        body,
        grid=(N_COMM // COMM_BLK,),
        in_specs=[
            pl.BlockSpec((COMM_BLK, 128), lambda i: (i, 0)),
            pl.BlockSpec((COMM_BLK, 16), lambda i: (i, 0)),
            pl.BlockSpec((COMM_BLK, 128), lambda i: (i, 0)),
            pl.BlockSpec((COMM_BLK, 128), lambda i: (i, 0)),
            pl.BlockSpec((COMM_BLK, 16), lambda i: (i, 0)),
            pl.BlockSpec((COMM_BLK, 128), lambda i: (i, 0)),
            _full((256, 128)), _full((1, 128)),
            _full((128, 1)), _full((1, 1)),
        ],
        out_specs=pl.BlockSpec((COMM_BLK, 1), lambda i: (i, 0)),
        out_shape=jax.ShapeDtypeStruct((N_COMM, 1), F32),
    )(s1, c1, m1, s2, c2, m2, wl1, bl1, wl2, bl2)


@functools.lru_cache(maxsize=None)
def _sc_gather_add():
    return pl.kernel(
        _sc_gather_add_body,
        mesh=_sc_mesh(),
        compiler_params=pltpu.CompilerParams(needs_layout_passes=False),
        out_type=(
            jax.ShapeDtypeStruct((E_PAD, 128), F32),
            jax.ShapeDtypeStruct((E_PAD,), I32),
        ),
        scratch_types=[
            pltpu.VMEM((GA_C,), I32),
            pltpu.VMEM((GA_C,), I32),
            pltpu.VMEM((GA_C, 128), F32),
            pltpu.VMEM((GA_C, 128), F32),
            pltpu.VMEM((GA_C,), I32),
            pltpu.SemaphoreType.DMA,
            pltpu.SemaphoreType.DMA,
        ],
    )


def _sc_gather_add_body(a_hbm, b_hbm, src_hbm, dst_hbm, z_out, key_out,
                   src_v, dst_v, bufa, bufb, key_v, sem1, sem2):
    base0 = _wid() * EPT

    def chunk(ci, carry):
        base = base0 + ci * GA_C
        pltpu.sync_copy(src_hbm.at[pl.ds(base, GA_C)], src_v)
        pltpu.sync_copy(dst_hbm.at[pl.ds(base, GA_C)], dst_v)
        cpa = pltpu.async_copy(a_hbm.at[src_v], bufa, sem1)
        cpb = pltpu.async_copy(b_hbm.at[dst_v], bufb, sem2)
        cpa.wait()
        cpb.wait()

        def addrow(i, c):
            r = i >> 3
            col = (i & 7) * 16
            sl = pl.ds(col, 16)
            bufa[r, sl] = bufa[r, sl] + bufb[r, sl]
            return c

        lax.fori_loop(0, GA_C * 8, addrow, 0)

        def keyvec(v, c):
            sl = pl.ds(v * 16, 16)
            sv = src_v[sl]
            dv = dst_v[sl]
            key_v[sl] = (dv << 16) | sv
            return c

        lax.fori_loop(0, GA_C // 16, keyvec, 0)
        pltpu.sync_copy(bufa, z_out.at[pl.ds(base, GA_C)])
        pltpu.sync_copy(key_v, key_out.at[pl.ds(base, GA_C)])
        return carry

    lax.fori_loop(0, GA_CHUNKS, chunk, 0)


@functools.lru_cache(maxsize=None)
def _sc_conv():
    return pl.kernel(
        _sc_conv_body,
        mesh=_sc_mesh(),
        compiler_params=pltpu.CompilerParams(needs_layout_passes=False),
        out_type=jax.ShapeDtypeStruct((N_NODES_ACC, 128), F32),
        scratch_types=[
            pltpu.VMEM((CK,), I32),
            pltpu.VMEM((CK + 16,), I32),
            pltpu.VMEM((CK + 16,), I32),
            pltpu.VMEM((CK + 16,), I32),
            pltpu.VMEM((SB, 128), F32),
            pltpu.VMEM((SB, 128), F32),
            pltpu.VMEM((RPV, 128), F32),
            pltpu.SemaphoreType.DMA,
            pltpu.SemaphoreType.DMA,
        ],
    )


def _sc_conv_body(hm, gt, keys, agg_out,
                  key_v, qsrc, qc, qeid, bufh, bufg, acc, sem1, sem2):
    w = _wid()

    z16 = jnp.zeros((16,), I32)

    def zeroq(i, c):
        sl = pl.ds(i * 16, 16)
        qsrc[sl] = z16
        qc[sl] = z16
        qeid[sl] = z16
        return c

    lax.fori_loop(0, (CK + 16) // 16, zeroq, 0)

    for p in range(2):
        vt = w * 2 + p
        lo = vt * RPV
        hi = lo + RPV

        zf = jnp.zeros((16,), F32)

        def zacc(i, c):
            acc[i >> 3, pl.ds((i & 7) * 16, 16)] = zf
            return c

        lax.fori_loop(0, RPV * 8, zacc, 0)

        def chunk(ci, carry):
            pltpu.sync_copy(keys.at[pl.ds(ci * CK, CK)], key_v)

            def scan(v, qn):
                sl = pl.ds(v * 16, 16)
                kv = key_v[sl]
                dv = lax.shift_right_logical(kv, 16)
                sv = kv & 0xFFFF
                gidx = ci * CK + v * 16 + lax.iota(I32, 16)
                m = (dv >= lo) & (dv < hi) & (gidx < E)
                plsc.store_compressed(qsrc.at[pl.ds(qn, 16)], sv, mask=m)
                plsc.store_compressed(qc.at[pl.ds(qn, 16)], dv - lo, mask=m)
                plsc.store_compressed(qeid.at[pl.ds(qn, 16)], gidx, mask=m)
                return qn + jnp.sum(m.astype(I32))

            qn = lax.fori_loop(0, CK // 16, scan, 0)
            nb = (qn + SB - 1) // SB

            def drain(b, c):
                cph = pltpu.async_copy(hm.at[qsrc.at[pl.ds(b * SB, SB)]],
                                       bufh, sem1)
                cpg = pltpu.async_copy(gt.at[qeid.at[pl.ds(b * SB, SB)]],
                                       bufg, sem2)
                cph.wait()
                cpg.wait()
                nj = jnp.minimum(SB, qn - b * SB)

                def rmw(j, cc):
                    ci_ = qc[pl.ds(b * SB + j, 16)][0]
                    for kk in range(8):
                        sl = pl.ds(kk * 16, 16)
                        acc[ci_, sl] = acc[ci_, sl] + bufh[j, sl] * bufg[j, sl]
                    return cc

                lax.fori_loop(0, nj, rmw, 0)
                return c

            lax.fori_loop(0, nb, drain, 0)
            return carry

        lax.fori_loop(0, CONV_CHUNKS, chunk, 0)
        pltpu.sync_copy(acc, agg_out.at[pl.ds(lo, RPV)])


@functools.lru_cache(maxsize=None)
def _sc_pool():
    return pl.kernel(
        _sc_pool_body,
        mesh=_sc_mesh(),
        compiler_params=pltpu.CompilerParams(needs_layout_passes=False),
        out_type=(
            jax.ShapeDtypeStruct((N_COMM_ACC, 128), F32),
            jax.ShapeDtypeStruct((N_COMM_ACC, 16), F32),
            jax.ShapeDtypeStruct((N_COMM_ACC, 128), F32),
        ),
        scratch_types=[
            pltpu.VMEM((PK,), I32),
            pltpu.VMEM((MK,), I32),
            pltpu.VMEM((MK,), I32),
            pltpu.VMEM((PK + 16,), I32),
            pltpu.VMEM((PK + 16,), I32),
            pltpu.VMEM((SBP, 128), F32),
            pltpu.VMEM((CPT, 128), F32),
            pltpu.VMEM((CPT, 16), F32),
            pltpu.VMEM((CPT, 128), F32),
            pltpu.SemaphoreType.DMA,
        ],
    )


def _sc_pool_body(h_hbm, comm_hbm, mcn_hbm, mci_hbm, sum_out, cnt_out, max_out,
             ids_v, ids2_v, rows2_v, qrow, qc, bufh, acc_s, acc_c, acc_m, sem):
    w = _wid()
    clo = w * CPT
    chi = clo + CPT

    z16 = jnp.zeros((16,), I32)

    def zeroq(i, c):
        sl = pl.ds(i * 16, 16)
        qrow[sl] = z16
        qc[sl] = z16
        return c

    lax.fori_loop(0, (PK + 16) // 16, zeroq, 0)

    zf = jnp.zeros((16,), F32)

    def zacc(i, c):
        acc_s[i >> 3, pl.ds((i & 7) * 16, 16)] = zf
        acc_m[i >> 3, pl.ds((i & 7) * 16, 16)] = zf
        return c

    lax.fori_loop(0, CPT * 8, zacc, 0)

    def zcnt(i, c):
        acc_c[i, pl.ds(0, 16)] = zf
        return c

    lax.fori_loop(0, CPT, zcnt, 0)

    ones = jnp.full((16,), 1.0, F32)

    def drain(qn):
        nb = (qn + SBP - 1) // SBP

        def dbody(b, c):
            cph = pltpu.async_copy(h_hbm.at[qrow.at[pl.ds(b * SBP, SBP)]],
                                   bufh, sem)
            cph.wait()
            nj = jnp.minimum(SBP, qn - b * SBP)

            def rmw(j, cc):
                ci_ = qc[pl.ds(b * SBP + j, 16)][0]
                for kk in range(8):
                    sl = pl.ds(kk * 16, 16)
                    v = bufh[j, sl]
                    acc_s[ci_, sl] = acc_s[ci_, sl] + v
                    acc_m[ci_, sl] = jnp.maximum(acc_m[ci_, sl], v)
                acc_c[ci_, pl.ds(0, 16)] = acc_c[ci_, pl.ds(0, 16)] + ones
                return cc

            lax.fori_loop(0, nj, rmw, 0)
            return c

        lax.fori_loop(0, nb, dbody, 0)

    def nchunk(ci, carry):
        pltpu.sync_copy(comm_hbm.at[pl.ds(ci * PK, PK)], ids_v)

        def scan(v, qn):
            sl = pl.ds(v * 16, 16)
            cv = ids_v[sl]
            gidx = ci * PK + v * 16 + lax.iota(I32, 16)
            m = (cv >= clo) & (cv < chi) & (gidx < N_NODES)
            plsc.store_compressed(qrow.at[pl.ds(qn, 16)], gidx, mask=m)
            plsc.store_compressed(qc.at[pl.ds(qn, 16)], cv - clo, mask=m)
            return qn + jnp.sum(m.astype(I32))

        qn = lax.fori_loop(0, PK // 16, scan, 0)
        drain(qn)
        return carry

    lax.fori_loop(0, NCHUNK_P, nchunk, 0)

    def mchunk(ci, carry):
        pltpu.sync_copy(mci_hbm.at[pl.ds(ci * MK, MK)], ids2_v)
        pltpu.sync_copy(mcn_hbm.at[pl.ds(ci * MK, MK)], rows2_v)

        def scan(v, qn):
            sl = pl.ds(v * 16, 16)
            cv = ids2_v[sl]
            rv = rows2_v[sl]
            gidx = ci * MK + v * 16 + lax.iota(I32, 16)
            m = (cv >= clo) & (cv < chi) & (gidx < N_MULTI)
            plsc.store_compressed(qrow.at[pl.ds(qn, 16)], rv, mask=m)
            plsc.store_compressed(qc.at[pl.ds(qn, 16)], cv - clo, mask=m)
            return qn + jnp.sum(m.astype(I32))

        qn = lax.fori_loop(0, MK // 16, scan, 0)
        drain(qn)
        return carry

    lax.fori_loop(0, MCHUNK_P, mchunk, 0)

    pltpu.sync_copy(acc_s, sum_out.at[pl.ds(clo, CPT)])
    pltpu.sync_copy(acc_c, cnt_out.at[pl.ds(clo, CPT)])
    pltpu.sync_copy(acc_m, max_out.at[pl.ds(clo, CPT)])


def kernel(x, edge_index, edge_attr, community, multi_community_nodes,
           multi_community_index, adj_inter, adj_intra, edge_attr_inter,
           edge_attr_intra, params):
    p = params

    def b2d(name):
        return p[name + "_b"].reshape(1, -1)

    pad_e = E_PAD - E
    zpad = jnp.zeros((pad_e,), I32)
    src_i = jnp.concatenate([adj_inter[0].astype(I32), zpad])
    dst_i = jnp.concatenate([adj_inter[1].astype(I32), zpad])
    src_a = jnp.concatenate([adj_intra[0].astype(I32), zpad])
    dst_a = jnp.concatenate([adj_intra[1].astype(I32), zpad])

    comm_p = jnp.concatenate([community.astype(I32),
                              jnp.zeros((NSCAN - N_NODES,), I32)])
    mcn_p = jnp.concatenate([multi_community_nodes.astype(I32),
                             jnp.zeros((MSCAN - N_MULTI,), I32)])
    mci_p = jnp.concatenate([multi_community_index.astype(I32),
                             jnp.zeros((MSCAN - N_MULTI,), I32)])

    h0 = _embed_call(x, p["emb1_w"], b2d("emb1"), p["emb2_w"], b2d("emb2"),
                     p["emb3_w"], b2d("emb3"))

    zb = jnp.zeros((1, 128), F32)
    (a1, b1, a2, b2, mi1, ri1, ma1, ra1) = _node_mats_call(
        h0,
        [(p["el1_n_w"][:128], b2d("el1_n")), (p["el1_n_w"][128:], zb),
         (p["el2_n_w"][:128], b2d("el2_n")), (p["el2_n_w"][128:], zb),
         (p["ci1_m_w"], b2d("ci1_m")), (p["ci1_r_w"], b2d("ci1_r")),
         (p["ca1_m_w"], b2d("ca1_m")), (p["ca1_r_w"], b2d("ca1_r"))],
    )

    ga = _sc_gather_add()
    z0_i, keys_i = ga(a1, b1, src_i, dst_i)
    z0_a, keys_a = ga(a2, b2, src_a, dst_a)

    g1i, g2i = _edge_g_call(
        z0_i, edge_attr_inter, p["ee1_w"], b2d("ee1"),
        p["el1_e_w"], b2d("el1_e"),
        p["ci1_g_w"], b2d("ci1_g"), p["ci2_g_w"], b2d("ci2_g"))
    g1a, g2a = _edge_g_call(
        z0_a, edge_attr_intra, p["ee2_w"], b2d("ee2"),
        p["el2_e_w"], b2d("el2_e"),
        p["ca1_g_w"], b2d("ca1_g"), p["ca2_g_w"], b2d("ca2_g"))

    conv = _sc_conv()
    agg_i1 = conv(mi1, g1i, keys_i)
    agg_a1 = conv(ma1, g1a, keys_a)

    (h1, mi2, ri2, ma2, ra2) = _combine_call(
        ri1, agg_i1, ra1, agg_a1,
        [(p["ci2_m_w"], b2d("ci2_m")), (p["ci2_r_w"], b2d("ci2_r")),
         (p["ca2_m_w"], b2d("ca2_m")), (p["ca2_r_w"], b2d("ca2_r"))],
    )

    s1, c1, m1 = _sc_pool()(h1, comm_p, mcn_p, mci_p)

    agg_i2 = conv(mi2, g2i, keys_i)
    agg_a2 = conv(ma2, g2a, keys_a)

    (h2,) = _combine_call(ri2, agg_i2, ra2, agg_a2, [])

    s2, c2, m2 = _sc_pool()(h2, comm_p, mcn_p, mci_p)

    out = _final_call(s1, c1, m1, s2, c2, m2,
                      p["lin1_w"], b2d("lin1"), p["lin2_w"], b2d("lin2"))
    return out.reshape(N_COMM)


# speedup vs baseline: 3.0227x; 3.0227x over previous
import functools

import jax
import jax.numpy as jnp
from jax import lax
from jax.experimental import pallas as pl
from jax.experimental.pallas import tpu as pltpu
from jax.experimental.pallas import tpu_sc as plsc

F32 = jnp.float32
I32 = jnp.int32

N_NODES = 50000
N_COMM = 5000
E = 400000
N_MULTI = 10000

NW = 32
E_PAD = 401408
EPT = E_PAD // NW
GA_C = 256
GA_CHUNKS = EPT // GA_C

NPS = 6272
N_NODES_ACC = 8 * NPS
SPS = 4
RWT = NPS // 16
EPS = E_PAD // 16
CK = 3136
CONV_CHUNKS = EPS // CK
SB = 192
ZR = 56
CONV_SENT = 60000 << 16

CPT = 160
N_COMM_ACC = NW * CPT
NPOOL = 60416
PK = 3776
POOL_CHUNKS = NPOOL // PK
POOL_SENT = 8192 << 16
SBP = 256

NODE_BLK = 2000
EDGE_BLK = 2000
COMM_BLK = 1000

@functools.lru_cache(maxsize=None)
def _sc_mesh():
    return plsc.VectorSubcoreMesh(core_axis_name="c", subcore_axis_name="s")


def _wid():
    return lax.axis_index("s") * 2 + lax.axis_index("c")


def _full(shape):
    nd = len(shape)
    return pl.BlockSpec(shape, lambda i: (0,) * nd)


def _embed_call(x, w1, b1, w2, b2, w3, b3):
    def body(x_ref, w1r, b1r, w2r, b2r, w3r, b3r, h_ref):
        xb = x_ref[...]
        x1 = jax.nn.relu(xb[:, :8] @ w1r[...] + b1r[...])
        x2 = jax.nn.relu(xb[:, 8:] @ w2r[...] + b2r[...])
        hh = jnp.concatenate([x1, x2], axis=1)
        h_ref[...] = jax.nn.relu(hh @ w3r[...] + b3r[...])

    return pl.pallas_call(
        body,
        grid=(N_NODES // NODE_BLK,),
        in_specs=[
            pl.BlockSpec((NODE_BLK, 20), lambda i: (i, 0)),
            _full((8, 64)), _full((1, 64)),
            _full((12, 64)), _full((1, 64)),
            _full((128, 128)), _full((1, 128)),
        ],
        out_specs=pl.BlockSpec((NODE_BLK, 128), lambda i: (i, 0)),
        out_shape=jax.ShapeDtypeStruct((N_NODES, 128), F32),
    )(x, w1, b1, w2, b2, w3, b3)


def _node_mats_call(h, wbs):
    k = len(wbs)

    def body(*refs):
        h_ref = refs[0]
        w_refs = refs[1:1 + k]
        b_refs = refs[1 + k:1 + 2 * k]
        out_refs = refs[1 + 2 * k:]
        hb = h_ref[...]
        for j in range(k):
            out_refs[j][...] = hb @ w_refs[j][...] + b_refs[j][...]

    out_specs = [pl.BlockSpec((NODE_BLK, 128), lambda i: (i, 0))] * k
    out_shape = [jax.ShapeDtypeStruct((N_NODES, 128), F32)] * k
    args = [h] + [w for (w, _) in wbs] + [b for (_, b) in wbs]
    return pl.pallas_call(
        body,
        grid=(N_NODES // NODE_BLK,),
        in_specs=[pl.BlockSpec((NODE_BLK, 128), lambda i: (i, 0))]
        + [_full((128, 128))] * k + [_full((1, 128))] * k,
        out_specs=out_specs,
        out_shape=out_shape,
    )(*args)


def _edge_g_call(z0, ea, wee, bee, wel, bel, wg1, bg1, wg2, bg2):
    def body(z_ref, ea_ref, weer, beer, welr, belr, wg1r, bg1r, wg2r, bg2r,
             o1, o2):
        e2 = jax.nn.relu(ea_ref[...] @ weer[...] + beer[...])
        e1 = e2 @ welr[...] + belr[...]
        mask = jnp.tanh(z_ref[...] + e1)
        o1[...] = jax.nn.sigmoid(mask @ wg1r[...] + bg1r[...])
        o2[...] = jax.nn.sigmoid(mask @ wg2r[...] + bg2r[...])

    eo = [pl.BlockSpec((EDGE_BLK, 128), lambda i: (i, 0))] * 2
    es = [jax.ShapeDtypeStruct((E, 128), F32)] * 2
    return pl.pallas_call(
        body,
        grid=(E // EDGE_BLK,),
        in_specs=[
            pl.BlockSpec((EDGE_BLK, 128), lambda i: (i, 0)),
            pl.BlockSpec((EDGE_BLK, 16), lambda i: (i, 0)),
            _full((16, 64)), _full((1, 64)),
            _full((64, 128)), _full((1, 128)),
            _full((128, 128)), _full((1, 128)),
            _full((128, 128)), _full((1, 128)),
        ],
        out_specs=eo,
        out_shape=es,
    )(z0, ea, wee, bee, wel, bel, wg1, bg1, wg2, bg2)


def _combine_call(ri, ai, ra, aa, wbs):
    k = len(wbs)

    def body(*refs):
        rir, air, rar, aar = refs[:4]
        w_refs = refs[4:4 + k]
        b_refs = refs[4 + k:4 + 2 * k]
        outs = refs[4 + 2 * k:]
        hb = jax.nn.relu(rir[...] + air[...]) + jax.nn.relu(rar[...] + aar[...])
        outs[0][...] = hb
        for j in range(k):
            outs[1 + j][...] = hb @ w_refs[j][...] + b_refs[j][...]

    out_specs = [pl.BlockSpec((NODE_BLK, 128), lambda i: (i, 0))] * (1 + k)
    out_shape = [jax.ShapeDtypeStruct((N_NODES, 128), F32)] * (1 + k)

    args = [ri, ai, ra, aa] + [w for (w, _) in wbs] + [b for (_, b) in wbs]
    return pl.pallas_call(
        body,
        grid=(N_NODES // NODE_BLK,),
        in_specs=[pl.BlockSpec((NODE_BLK, 128), lambda i: (i, 0))] * 4
        + [_full((128, 128))] * k + [_full((1, 128))] * k,
        out_specs=out_specs,
        out_shape=out_shape,
    )(*args)


def _final_call(s1, c1, m1, s2, c2, m2, wl1, bl1, wl2, bl2):
    def body(s1r, c1r, m1r, s2r, c2r, m2r, w1r, b1r, w2r, b2r, out_ref):
        mean1 = s1r[...] / jnp.maximum(c1r[...][:, :1], 1.0)
        mean2 = s2r[...] / jnp.maximum(c2r[...][:, :1], 1.0)
        g = (jnp.concatenate([mean1, m1r[...]], axis=1)
             + jnp.concatenate([mean2, m2r[...]], axis=1))
        gg = jax.nn.relu(g @ w1r[...] + b1r[...])
        out_ref[...] = gg @ w2r[...] + b2r[...]

    return pl.pallas_call(
        body,
        grid=(N_COMM // COMM_BLK,),
        in_specs=[
            pl.BlockSpec((COMM_BLK, 128), lambda i: (i, 0)),
            pl.BlockSpec((COMM_BLK, 16), lambda i: (i, 0)),
            pl.BlockSpec((COMM_BLK, 128), lambda i: (i, 0)),
            pl.BlockSpec((COMM_BLK, 128), lambda i: (i, 0)),
            pl.BlockSpec((COMM_BLK, 16), lambda i: (i, 0)),
            pl.BlockSpec((COMM_BLK, 128), lambda i: (i, 0)),
            _full((256, 128)), _full((1, 128)),
            _full((128, 1)), _full((1, 1)),
        ],
        out_specs=pl.BlockSpec((COMM_BLK, 1), lambda i: (i, 0)),
        out_shape=jax.ShapeDtypeStruct((N_COMM, 1), F32),
    )(s1, c1, m1, s2, c2, m2, wl1, bl1, wl2, bl2)


def _pack_pool_keys_call(comm2d, mci2d, mcn2d):
    def nbody(c_ref, o_ref):
        r = jax.lax.broadcasted_iota(I32, (25, 2000), 0)
        cc = jax.lax.broadcasted_iota(I32, (25, 2000), 1)
        o_ref[...] = (c_ref[...] << 16) | (r * 2000 + cc)

    def mbody(ci_ref, cn_ref, o_ref):
        o_ref[...] = (ci_ref[...] << 16) | cn_ref[...]

    kn = pl.pallas_call(
        nbody,
        grid=(1,),
        in_specs=[pl.BlockSpec((25, 2000), lambda i: (0, 0))],
        out_specs=pl.BlockSpec((25, 2000), lambda i: (0, 0)),
        out_shape=jax.ShapeDtypeStruct((25, 2000), I32),
    )(comm2d)
    km = pl.pallas_call(
        mbody,
        grid=(1,),
        in_specs=[pl.BlockSpec((5, 2000), lambda i: (0, 0))] * 2,
        out_specs=pl.BlockSpec((5, 2000), lambda i: (0, 0)),
        out_shape=jax.ShapeDtypeStruct((5, 2000), I32),
    )(mci2d, mcn2d)
    return kn, km


@functools.lru_cache(maxsize=None)
def _sc_gather_add():
    return pl.kernel(
        _sc_gather_add_body,
        mesh=_sc_mesh(),
        compiler_params=pltpu.CompilerParams(needs_layout_passes=False),
        out_type=(
            jax.ShapeDtypeStruct((E_PAD, 128), F32),
            jax.ShapeDtypeStruct((E_PAD,), I32),
        ),
        scratch_types=[
            pltpu.VMEM((GA_C,), I32),
            pltpu.VMEM((GA_C,), I32),
            pltpu.VMEM((GA_C, 128), F32),
            pltpu.VMEM((GA_C, 128), F32),
            pltpu.VMEM((GA_C,), I32),
            pltpu.SemaphoreType.DMA,
            pltpu.SemaphoreType.DMA,
        ],
    )


def _sc_gather_add_body(a_hbm, b_hbm, src_hbm, dst_hbm, z_out, key_out,
                   src_v, dst_v, bufa, bufb, key_v, sem1, sem2):
    base0 = _wid() * EPT

    def chunk(ci, carry):
        base = base0 + ci * GA_C
        pltpu.sync_copy(src_hbm.at[pl.ds(base, GA_C)], src_v)
        pltpu.sync_copy(dst_hbm.at[pl.ds(base, GA_C)], dst_v)
        cpa = pltpu.async_copy(a_hbm.at[src_v], bufa, sem1)
        cpb = pltpu.async_copy(b_hbm.at[dst_v], bufb, sem2)
        cpa.wait()
        cpb.wait()

        def addrow(r, c):
            for kk in range(8):
                sl = pl.ds(kk * 16, 16)
                bufa[r, sl] = bufa[r, sl] + bufb[r, sl]
            return c

        lax.fori_loop(0, GA_C, addrow, 0)

        iota16 = lax.iota(I32, 16)

        def keyvec(v, c):
            sl = pl.ds(v * 16, 16)
            sv = src_v[sl]
            dv = dst_v[sl]
            gi = base + v * 16 + iota16
            key_v[sl] = jnp.where(gi < E, (dv << 16) | sv,
                                  jnp.full((16,), CONV_SENT, I32))
            return c

        lax.fori_loop(0, GA_C // 16, keyvec, 0)
        pltpu.sync_copy(bufa, z_out.at[pl.ds(base, GA_C)])
        pltpu.sync_copy(key_v, key_out.at[pl.ds(base, GA_C)])
        return carry

    lax.fori_loop(0, GA_CHUNKS, chunk, 0)


@functools.lru_cache(maxsize=None)
def _sc_conv():
    return pl.kernel(
        _sc_conv_body,
        mesh=_sc_mesh(),
        compiler_params=pltpu.CompilerParams(needs_layout_passes=False),
        out_type=jax.ShapeDtypeStruct((N_NODES_ACC, 128), F32),
        scratch_types=[
            pltpu.VMEM((CK,), I32),
            pltpu.VMEM((CK + 16 + SB,), I32),
            pltpu.VMEM((CK + 16 + SB,), I32),
            pltpu.VMEM((CK + 16 + SB,), I32),
            pltpu.VMEM((SB, 128), F32),
            pltpu.VMEM((SB, 128), F32),
            pltpu.VMEM((ZR, 128), F32),
            pltpu.VMEM_SHARED((NPS + 16, 128), F32),
            pltpu.SemaphoreType.DMA,
            pltpu.SemaphoreType.DMA,
        ],
    )


def _sc_conv_body(hm, gt, keys, agg_out,
                  key_v, qsrc, qdst, qeid, bufh, bufg, zbuf,
                  shared, sem1, sem2):
    c = lax.axis_index("c")
    sct = lax.axis_index("s")
    dummy = NPS + sct

    z16i = jnp.zeros((16,), I32)

    def zeroq(i, cc):
        sl = pl.ds(i * 16, 16)
        qsrc[sl] = z16i
        qdst[sl] = z16i
        qeid[sl] = z16i
        return cc

    lax.fori_loop(0, (CK + 16 + SB) // 16, zeroq, 0)

    zf = jnp.zeros((16,), F32)

    def zzb(i, cc):
        zbuf[i >> 3, pl.ds((i & 7) * 16, 16)] = zf
        return cc

    lax.fori_loop(0, ZR * 8, zzb, 0)

    iota16 = lax.iota(I32, 16)
    dumv = jnp.zeros((16,), I32) + dummy

    def qpass(q, carry):
        section = c * SPS + q
        lo = section * NPS
        hi = lo + NPS

        def clr(k, cc):
            pltpu.sync_copy(zbuf, shared.at[pl.ds(sct * RWT + k * ZR, ZR)])
            return cc

        lax.fori_loop(0, RWT // ZR, clr, 0)
        pltpu.sync_copy(zbuf.at[pl.ds(0, 16)], shared.at[pl.ds(NPS, 16)])
        plsc.subcore_barrier()

        def chunk(ci, carry2):
            base = sct * EPS + ci * CK
            pltpu.sync_copy(keys.at[pl.ds(base, CK)], key_v)

            def scan(v, qn):
                sl = pl.ds(v * 16, 16)
                kv = key_v[sl]
                dv = lax.shift_right_logical(kv, 16)
                m = (dv >= lo) & (dv < hi)
                plsc.store_compressed(qsrc.at[pl.ds(qn, 16)], kv & 0xFFFF,
                                      mask=m)
                plsc.store_compressed(qdst.at[pl.ds(qn, 16)], dv - lo, mask=m)
                plsc.store_compressed(qeid.at[pl.ds(qn, 16)],
                                      base + v * 16 + iota16, mask=m)
                return qn + jnp.sum(m.astype(I32))

            qn = lax.fori_loop(0, CK // 16, scan, 0)
            for t in range(SB // 16):
                qdst[pl.ds(qn + t * 16, 16)] = dumv
            nb = (qn + SB - 1) // SB

            def drain(b, cc):
                cph = pltpu.async_copy(hm.at[qsrc.at[pl.ds(b * SB, SB)]],
                                       bufh, sem1)
                cpg = pltpu.async_copy(gt.at[qeid.at[pl.ds(b * SB, SB)]],
                                       bufg, sem2)
                cph.wait()
                cpg.wait()

                def mul(r, c3):
                    for kk in range(8):
                        slm = pl.ds(kk * 16, 16)
                        bufh[r, slm] = bufh[r, slm] * bufg[r, slm]
                    return c3

                lax.fori_loop(0, SB, mul, 0)
                pltpu.sync_copy(bufh, shared.at[qdst.at[pl.ds(b * SB, SB)]],
                                add=True)
                return cc

            lax.fori_loop(0, nb, drain, 0)
            return carry2

        lax.fori_loop(0, CONV_CHUNKS, chunk, 0)
        plsc.subcore_barrier()
        pltpu.sync_copy(shared.at[pl.ds(sct * RWT, RWT)],
                        agg_out.at[pl.ds(lo + sct * RWT, RWT)])
        plsc.subcore_barrier()
        return carry

    lax.fori_loop(0, SPS, qpass, 0)


@functools.lru_cache(maxsize=None)
def _sc_pool():
    return pl.kernel(
        _sc_pool_body,
        mesh=_sc_mesh(),
        compiler_params=pltpu.CompilerParams(needs_layout_passes=False),
        out_type=(
            jax.ShapeDtypeStruct((N_COMM_ACC, 128), F32),
            jax.ShapeDtypeStruct((N_COMM_ACC, 16), F32),
            jax.ShapeDtypeStruct((N_COMM_ACC, 128), F32),
        ),
        scratch_types=[
            pltpu.VMEM((PK,), I32),
            pltpu.VMEM((PK + 16,), I32),
            pltpu.VMEM((PK + 16,), I32),
            pltpu.VMEM((SBP, 128), F32),
            pltpu.VMEM((CPT, 128), F32),
            pltpu.VMEM((CPT, 16), F32),
            pltpu.VMEM((CPT, 128), F32),
            pltpu.SemaphoreType.DMA,
        ],
    )


def _sc_pool_body(h_hbm, pkeys, sum_out, cnt_out, max_out,
                  key_v, qrow, qc, bufh, acc_s, acc_c, acc_m, sem):
    w = _wid()
    clo = w * CPT
    chi = clo + CPT

    z16 = jnp.zeros((16,), I32)

    def zeroq(i, c):
        sl = pl.ds(i * 16, 16)
        qrow[sl] = z16
        qc[sl] = z16
        return c

    lax.fori_loop(0, (PK + 16) // 16, zeroq, 0)

    zf = jnp.zeros((16,), F32)

    def zacc(i, c):
        acc_s[i >> 3, pl.ds((i & 7) * 16, 16)] = zf
        acc_m[i >> 3, pl.ds((i & 7) * 16, 16)] = zf
        return c

    lax.fori_loop(0, CPT * 8, zacc, 0)

    def zcnt(i, c):
        acc_c[i, pl.ds(0, 16)] = zf
        return c

    lax.fori_loop(0, CPT, zcnt, 0)

    ones = jnp.full((16,), 1.0, F32)

    def chunkf(ci, carry):
        pltpu.sync_copy(pkeys.at[pl.ds(ci * PK, PK)], key_v)

        def scan(v, qn):
            sl = pl.ds(v * 16, 16)
            kv = key_v[sl]
            cv = lax.shift_right_logical(kv, 16)
            m = (cv >= clo) & (cv < chi)
            plsc.store_compressed(qrow.at[pl.ds(qn, 16)], kv & 0xFFFF, mask=m)
            plsc.store_compressed(qc.at[pl.ds(qn, 16)], cv - clo, mask=m)
            return qn + jnp.sum(m.astype(I32))

        qn = lax.fori_loop(0, PK // 16, scan, 0)
        nb = (qn + SBP - 1) // SBP

        def dbody(b, c):
            cph = pltpu.async_copy(h_hbm.at[qrow.at[pl.ds(b * SBP, SBP)]],
                                   bufh, sem)
            cph.wait()
            nj = jnp.minimum(SBP, qn - b * SBP)

            def rmw(j, cc):
                ci_ = qc[pl.ds(b * SBP + j, 16)][0]
                for kk in range(8):
                    sl = pl.ds(kk * 16, 16)
                    v = bufh[j, sl]
                    acc_s[ci_, sl] = acc_s[ci_, sl] + v
                    acc_m[ci_, sl] = jnp.maximum(acc_m[ci_, sl], v)
                acc_c[ci_, pl.ds(0, 16)] = acc_c[ci_, pl.ds(0, 16)] + ones
                return cc

            lax.fori_loop(0, nj, rmw, 0)
            return c

        lax.fori_loop(0, nb, dbody, 0)
        return carry

    lax.fori_loop(0, POOL_CHUNKS, chunkf, 0)

    pltpu.sync_copy(acc_s, sum_out.at[pl.ds(clo, CPT)])
    pltpu.sync_copy(acc_c, cnt_out.at[pl.ds(clo, CPT)])
    pltpu.sync_copy(acc_m, max_out.at[pl.ds(clo, CPT)])


def kernel(x, edge_index, edge_attr, community, multi_community_nodes,
           multi_community_index, adj_inter, adj_intra, edge_attr_inter,
           edge_attr_intra, params):
    p = params

    def b2d(name):
        return p[name + "_b"].reshape(1, -1)

    pad_e = E_PAD - E
    zpad = jnp.zeros((pad_e,), I32)
    src_i = jnp.concatenate([adj_inter[0].astype(I32), zpad])
    dst_i = jnp.concatenate([adj_inter[1].astype(I32), zpad])
    src_a = jnp.concatenate([adj_intra[0].astype(I32), zpad])
    dst_a = jnp.concatenate([adj_intra[1].astype(I32), zpad])

    kn, km = _pack_pool_keys_call(
        community.astype(I32).reshape(25, 2000),
        multi_community_index.astype(I32).reshape(5, 2000),
        multi_community_nodes.astype(I32).reshape(5, 2000))
    pool_keys = jnp.concatenate([
        kn.reshape(N_NODES), km.reshape(N_MULTI),
        jnp.full((NPOOL - N_NODES - N_MULTI,), POOL_SENT, I32)])

    h0 = _embed_call(x, p["emb1_w"], b2d("emb1"), p["emb2_w"], b2d("emb2"),
                     p["emb3_w"], b2d("emb3"))

    zb = jnp.zeros((1, 128), F32)
    (a1, b1, a2, b2, mi1, ri1, ma1, ra1) = _node_mats_call(
        h0,
        [(p["el1_n_w"][:128], b2d("el1_n")), (p["el1_n_w"][128:], zb),
         (p["el2_n_w"][:128], b2d("el2_n")), (p["el2_n_w"][128:], zb),
         (p["ci1_m_w"], b2d("ci1_m")), (p["ci1_r_w"], b2d("ci1_r")),
         (p["ca1_m_w"], b2d("ca1_m")), (p["ca1_r_w"], b2d("ca1_r"))],
    )

    ga = _sc_gather_add()
    z0_i, keys_i = ga(a1, b1, src_i, dst_i)
    z0_a, keys_a = ga(a2, b2, src_a, dst_a)

    g1i, g2i = _edge_g_call(
        z0_i, edge_attr_inter, p["ee1_w"], b2d("ee1"),
        p["el1_e_w"], b2d("el1_e"),
        p["ci1_g_w"], b2d("ci1_g"), p["ci2_g_w"], b2d("ci2_g"))
    g1a, g2a = _edge_g_call(
        z0_a, edge_attr_intra, p["ee2_w"], b2d("ee2"),
        p["el2_e_w"], b2d("el2_e"),
        p["ca1_g_w"], b2d("ca1_g"), p["ca2_g_w"], b2d("ca2_g"))

    conv = _sc_conv()
    agg_i1 = conv(mi1, g1i, keys_i)
    agg_a1 = conv(ma1, g1a, keys_a)

    (h1, mi2, ri2, ma2, ra2) = _combine_call(
        ri1, agg_i1, ra1, agg_a1,
        [(p["ci2_m_w"], b2d("ci2_m")), (p["ci2_r_w"], b2d("ci2_r")),
         (p["ca2_m_w"], b2d("ca2_m")), (p["ca2_r_w"], b2d("ca2_r"))],
    )

    s1, c1, m1 = _sc_pool()(h1, pool_keys)

    agg_i2 = conv(mi2, g2i, keys_i)
    agg_a2 = conv(ma2, g2a, keys_a)

    (h2,) = _combine_call(ri2, agg_i2, ra2, agg_a2, [])

    s2, c2, m2 = _sc_pool()(h2, pool_keys)

    out = _final_call(s1, c1, m1, s2, c2, m2,
                      p["lin1_w"], b2d("lin1"), p["lin2_w"], b2d("lin2"))
    return out.reshape(N_COMM)


# speedup vs baseline: 4.8626x; 1.6087x over previous
import functools

import jax
import jax.numpy as jnp
from jax import lax
from jax.experimental import pallas as pl
from jax.experimental.pallas import tpu as pltpu
from jax.experimental.pallas import tpu_sc as plsc

F32 = jnp.float32
I32 = jnp.int32

N_NODES = 50000
N_COMM = 5000
E = 400000
N_MULTI = 10000

NW = 32
E_PAD = 401408
EPT = E_PAD // NW
GA_C = 256
GA_CHUNKS = EPT // GA_C

NPS = 6272
N_NODES_ACC = 8 * NPS
SPS = 4
RWT = NPS // 16
EPS = E_PAD // 16
CK = 1568
CONV_CHUNKS = EPS // CK
SB = 128
ZR = 56
CONV_SENT = 60000 << 16

CPT = 160
N_COMM_ACC = NW * CPT
NPOOL = 60416
PK = 1888
POOL_CHUNKS = NPOOL // PK
POOL_SENT = 8192 << 16
SBP = 64

NODE_BLK = 2000
EDGE_BLK = 2000
COMM_BLK = 1000

@functools.lru_cache(maxsize=None)
def _sc_mesh():
    return plsc.VectorSubcoreMesh(core_axis_name="c", subcore_axis_name="s")


def _wid():
    return lax.axis_index("s") * 2 + lax.axis_index("c")


def _full(shape):
    nd = len(shape)
    return pl.BlockSpec(shape, lambda i: (0,) * nd)


def _embed_call(x, w1, b1, w2, b2, w3, b3):
    def body(x_ref, w1r, b1r, w2r, b2r, w3r, b3r, h_ref):
        xb = x_ref[...]
        x1 = jax.nn.relu(xb[:, :8] @ w1r[...] + b1r[...])
        x2 = jax.nn.relu(xb[:, 8:] @ w2r[...] + b2r[...])
        hh = jnp.concatenate([x1, x2], axis=1)
        h_ref[...] = jax.nn.relu(hh @ w3r[...] + b3r[...])

    return pl.pallas_call(
        body,
        grid=(N_NODES // NODE_BLK,),
        in_specs=[
            pl.BlockSpec((NODE_BLK, 20), lambda i: (i, 0)),
            _full((8, 64)), _full((1, 64)),
            _full((12, 64)), _full((1, 64)),
            _full((128, 128)), _full((1, 128)),
        ],
        out_specs=pl.BlockSpec((NODE_BLK, 128), lambda i: (i, 0)),
        out_shape=jax.ShapeDtypeStruct((N_NODES, 128), F32),
    )(x, w1, b1, w2, b2, w3, b3)


def _node_mats_call(h, wbs):
    k = len(wbs)

    def body(*refs):
        h_ref = refs[0]
        w_refs = refs[1:1 + k]
        b_refs = refs[1 + k:1 + 2 * k]
        out_refs = refs[1 + 2 * k:]
        hb = h_ref[...]
        for j in range(k):
            out_refs[j][...] = hb @ w_refs[j][...] + b_refs[j][...]

    out_specs = [pl.BlockSpec((NODE_BLK, 128), lambda i: (i, 0))] * k
    out_shape = [jax.ShapeDtypeStruct((N_NODES, 128), F32)] * k
    args = [h] + [w for (w, _) in wbs] + [b for (_, b) in wbs]
    return pl.pallas_call(
        body,
        grid=(N_NODES // NODE_BLK,),
        in_specs=[pl.BlockSpec((NODE_BLK, 128), lambda i: (i, 0))]
        + [_full((128, 128))] * k + [_full((1, 128))] * k,
        out_specs=out_specs,
        out_shape=out_shape,
    )(*args)


def _edge_g_call(z0, ea, wee, bee, wel, bel, wg1, bg1, wg2, bg2):
    def body(z_ref, ea_ref, weer, beer, welr, belr, wg1r, bg1r, wg2r, bg2r,
             o1, o2):
        e2 = jax.nn.relu(ea_ref[...] @ weer[...] + beer[...])
        e1 = e2 @ welr[...] + belr[...]
        mask = jnp.tanh(z_ref[...] + e1)
        o1[...] = jax.nn.sigmoid(mask @ wg1r[...] + bg1r[...])
        o2[...] = jax.nn.sigmoid(mask @ wg2r[...] + bg2r[...])

    eo = [pl.BlockSpec((EDGE_BLK, 128), lambda i: (i, 0))] * 2
    es = [jax.ShapeDtypeStruct((E, 128), F32)] * 2
    return pl.pallas_call(
        body,
        grid=(E // EDGE_BLK,),
        in_specs=[
            pl.BlockSpec((EDGE_BLK, 128), lambda i: (i, 0)),
            pl.BlockSpec((EDGE_BLK, 16), lambda i: (i, 0)),
            _full((16, 64)), _full((1, 64)),
            _full((64, 128)), _full((1, 128)),
            _full((128, 128)), _full((1, 128)),
            _full((128, 128)), _full((1, 128)),
        ],
        out_specs=eo,
        out_shape=es,
    )(z0, ea, wee, bee, wel, bel, wg1, bg1, wg2, bg2)


def _combine_call(ri, ai, ra, aa, wbs):
    k = len(wbs)

    def body(*refs):
        rir, air, rar, aar = refs[:4]
        w_refs = refs[4:4 + k]
        b_refs = refs[4 + k:4 + 2 * k]
        outs = refs[4 + 2 * k:]
        hb = jax.nn.relu(rir[...] + air[...]) + jax.nn.relu(rar[...] + aar[...])
        outs[0][...] = hb
        for j in range(k):
            outs[1 + j][...] = hb @ w_refs[j][...] + b_refs[j][...]

    out_specs = [pl.BlockSpec((NODE_BLK, 128), lambda i: (i, 0))] * (1 + k)
    out_shape = [jax.ShapeDtypeStruct((N_NODES, 128), F32)] * (1 + k)

    args = [ri, ai, ra, aa] + [w for (w, _) in wbs] + [b for (_, b) in wbs]
    return pl.pallas_call(
        body,
        grid=(N_NODES // NODE_BLK,),
        in_specs=[pl.BlockSpec((NODE_BLK, 128), lambda i: (i, 0))] * 4
        + [_full((128, 128))] * k + [_full((1, 128))] * k,
        out_specs=out_specs,
        out_shape=out_shape,
    )(*args)


def _final_call(s1, c1, m1, s2, c2, m2, wl1, bl1, wl2, bl2):
    def body(s1r, c1r, m1r, s2r, c2r, m2r, w1r, b1r, w2r, b2r, out_ref):
        mean1 = s1r[...] / jnp.maximum(c1r[...][:, :1], 1.0)
        mean2 = s2r[...] / jnp.maximum(c2r[...][:, :1], 1.0)
        g = (jnp.concatenate([mean1, m1r[...]], axis=1)
             + jnp.concatenate([mean2, m2r[...]], axis=1))
        gg = jax.nn.relu(g @ w1r[...] + b1r[...])
        out_ref[...] = gg @ w2r[...] + b2r[...]

    return pl.pallas_call(
        body,
        grid=(N_COMM // COMM_BLK,),
        in_specs=[
            pl.BlockSpec((COMM_BLK, 128), lambda i: (i, 0)),
            pl.BlockSpec((COMM_BLK, 16), lambda i: (i, 0)),
            pl.BlockSpec((COMM_BLK, 128), lambda i: (i, 0)),
            pl.BlockSpec((COMM_BLK, 128), lambda i: (i, 0)),
            pl.BlockSpec((COMM_BLK, 16), lambda i: (i, 0)),
            pl.BlockSpec((COMM_BLK, 128), lambda i: (i, 0)),
            _full((256, 128)), _full((1, 128)),
            _full((128, 1)), _full((1, 1)),
        ],
        out_specs=pl.BlockSpec((COMM_BLK, 1), lambda i: (i, 0)),
        out_shape=jax.ShapeDtypeStruct((N_COMM, 1), F32),
    )(s1, c1, m1, s2, c2, m2, wl1, bl1, wl2, bl2)


def _pack_pool_keys_call(comm2d, mci2d, mcn2d):
    def nbody(c_ref, o_ref):
        r = jax.lax.broadcasted_iota(I32, (25, 2000), 0)
        cc = jax.lax.broadcasted_iota(I32, (25, 2000), 1)
        o_ref[...] = (c_ref[...] << 16) | (r * 2000 + cc)

    def mbody(ci_ref, cn_ref, o_ref):
        o_ref[...] = (ci_ref[...] << 16) | cn_ref[...]

    kn = pl.pallas_call(
        nbody,
        grid=(1,),
        in_specs=[pl.BlockSpec((25, 2000), lambda i: (0, 0))],
        out_specs=pl.BlockSpec((25, 2000), lambda i: (0, 0)),
        out_shape=jax.ShapeDtypeStruct((25, 2000), I32),
    )(comm2d)
    km = pl.pallas_call(
        mbody,
        grid=(1,),
        in_specs=[pl.BlockSpec((5, 2000), lambda i: (0, 0))] * 2,
        out_specs=pl.BlockSpec((5, 2000), lambda i: (0, 0)),
        out_shape=jax.ShapeDtypeStruct((5, 2000), I32),
    )(mci2d, mcn2d)
    return kn, km


@functools.lru_cache(maxsize=None)
def _sc_gather_add():
    return pl.kernel(
        _sc_gather_add_body,
        mesh=_sc_mesh(),
        compiler_params=pltpu.CompilerParams(needs_layout_passes=False),
        out_type=(
            jax.ShapeDtypeStruct((E_PAD, 128), F32),
            jax.ShapeDtypeStruct((E_PAD,), I32),
        ),
        scratch_types=[
            pltpu.VMEM((GA_C,), I32),
            pltpu.VMEM((GA_C,), I32),
            pltpu.VMEM((GA_C, 128), F32),
            pltpu.VMEM((GA_C, 128), F32),
            pltpu.VMEM((GA_C,), I32),
            pltpu.SemaphoreType.DMA,
            pltpu.SemaphoreType.DMA,
        ],
    )


def _sc_gather_add_body(a_hbm, b_hbm, src_hbm, dst_hbm, z_out, key_out,
                   src_v, dst_v, bufa, bufb, key_v, sem1, sem2):
    base0 = _wid() * EPT

    def chunk(ci, carry):
        base = base0 + ci * GA_C
        pltpu.sync_copy(src_hbm.at[pl.ds(base, GA_C)], src_v)
        pltpu.sync_copy(dst_hbm.at[pl.ds(base, GA_C)], dst_v)
        cpa = pltpu.async_copy(a_hbm.at[src_v], bufa, sem1)
        cpb = pltpu.async_copy(b_hbm.at[dst_v], bufb, sem2)
        cpa.wait()
        cpb.wait()

        def addrow(r, c):
            for kk in range(8):
                sl = pl.ds(kk * 16, 16)
                bufa[r, sl] = bufa[r, sl] + bufb[r, sl]
            return c

        lax.fori_loop(0, GA_C, addrow, 0)

        iota16 = lax.iota(I32, 16)

        def keyvec(v, c):
            sl = pl.ds(v * 16, 16)
            sv = src_v[sl]
            dv = dst_v[sl]
            gi = base + v * 16 + iota16
            key_v[sl] = jnp.where(gi < E, (dv << 16) | sv,
                                  jnp.full((16,), CONV_SENT, I32))
            return c

        lax.fori_loop(0, GA_C // 16, keyvec, 0)
        pltpu.sync_copy(bufa, z_out.at[pl.ds(base, GA_C)])
        pltpu.sync_copy(key_v, key_out.at[pl.ds(base, GA_C)])
        return carry

    lax.fori_loop(0, GA_CHUNKS, chunk, 0)


@functools.lru_cache(maxsize=None)
def _sc_conv():
    return pl.kernel(
        _sc_conv_body,
        mesh=_sc_mesh(),
        compiler_params=pltpu.CompilerParams(needs_layout_passes=False),
        out_type=jax.ShapeDtypeStruct((N_NODES_ACC, 128), F32),
        scratch_types=[
            pltpu.VMEM((CK,), I32),
            pltpu.VMEM((CK + 16 + SB,), I32),
            pltpu.VMEM((CK + 16 + SB,), I32),
            pltpu.VMEM((CK + 16 + SB,), I32),
            pltpu.VMEM((SB, 128), F32),
            pltpu.VMEM((SB, 128), F32),
            pltpu.VMEM((SB, 128), F32),
            pltpu.VMEM((ZR, 128), F32),
            pltpu.VMEM_SHARED((NPS + 16, 128), F32),
            pltpu.SemaphoreType.DMA,
            pltpu.SemaphoreType.DMA,
        ],
    )


def _sc_conv_body(hm, gt, keys, agg_out,
                  key_v, qsrc, qdst, qeid, bufh, bufg, msg, zbuf,
                  shared, sem1, sem2):
    c = lax.axis_index("c")
    sct = lax.axis_index("s")
    dummy = NPS + sct

    z16i = jnp.zeros((16,), I32)

    def zeroq(i, cc):
        sl = pl.ds(i * 16, 16)
        qsrc[sl] = z16i
        qdst[sl] = z16i
        qeid[sl] = z16i
        return cc

    lax.fori_loop(0, (CK + 16 + SB) // 16, zeroq, 0)

    zf = jnp.zeros((16,), F32)

    def zzb(i, cc):
        zbuf[i >> 3, pl.ds((i & 7) * 16, 16)] = zf
        return cc

    lax.fori_loop(0, ZR * 8, zzb, 0)

    iota16 = lax.iota(I32, 16)
    dumv = jnp.zeros((16,), I32) + dummy

    def qpass(q, carry):
        section = c * SPS + q
        lo = section * NPS
        hi = lo + NPS

        def clr(k, cc):
            pltpu.sync_copy(zbuf, shared.at[pl.ds(sct * RWT + k * ZR, ZR)])
            return cc

        lax.fori_loop(0, RWT // ZR, clr, 0)
        pltpu.sync_copy(zbuf.at[pl.ds(0, 16)], shared.at[pl.ds(NPS, 16)])
        plsc.subcore_barrier()

        def chunk(ci, carry2):
            base = sct * EPS + ci * CK
            pltpu.sync_copy(keys.at[pl.ds(base, CK)], key_v)

            def scan(v, qn):
                sl = pl.ds(v * 16, 16)
                kv = key_v[sl]
                dv = lax.shift_right_logical(kv, 16)
                m = (dv >= lo) & (dv < hi)
                plsc.store_compressed(qsrc.at[pl.ds(qn, 16)], kv & 0xFFFF,
                                      mask=m)
                plsc.store_compressed(qdst.at[pl.ds(qn, 16)], dv - lo, mask=m)
                plsc.store_compressed(qeid.at[pl.ds(qn, 16)],
                                      base + v * 16 + iota16, mask=m)
                return qn + jnp.sum(m.astype(I32))

            qn = lax.fori_loop(0, CK // 16, scan, 0)
            for t in range(SB // 16):
                qdst[pl.ds(qn + t * 16, 16)] = dumv
            nb = (qn + SB - 1) // SB

            def drain(b, cc):
                cph = pltpu.async_copy(hm.at[qsrc.at[pl.ds(b * SB, SB)]],
                                       bufh, sem1)
                cpg = pltpu.async_copy(gt.at[qeid.at[pl.ds(b * SB, SB)]],
                                       bufg, sem2)
                cph.wait()
                cpg.wait()

                def mul(r, c3):
                    for kk in range(8):
                        slm = pl.ds(kk * 16, 16)
                        msg[r, slm] = bufh[r, slm] * bufg[r, slm]
                    return c3

                lax.fori_loop(0, SB, mul, 0)
                pltpu.sync_copy(msg, shared.at[qdst.at[pl.ds(b * SB, SB)]],
                                add=True)
                return cc

            lax.fori_loop(0, nb, drain, 0)
            return carry2

        lax.fori_loop(0, CONV_CHUNKS, chunk, 0)
        plsc.subcore_barrier()
        pltpu.sync_copy(shared.at[pl.ds(sct * RWT, RWT)],
                        agg_out.at[pl.ds(lo + sct * RWT, RWT)])
        plsc.subcore_barrier()
        return carry

    lax.fori_loop(0, SPS, qpass, 0)


@functools.lru_cache(maxsize=None)
def _sc_pool():
    return pl.kernel(
        _sc_pool_body,
        mesh=_sc_mesh(),
        compiler_params=pltpu.CompilerParams(needs_layout_passes=False),
        out_type=(
            jax.ShapeDtypeStruct((N_COMM_ACC, 128), F32),
            jax.ShapeDtypeStruct((N_COMM_ACC, 16), F32),
            jax.ShapeDtypeStruct((N_COMM_ACC, 128), F32),
        ),
        scratch_types=[
            pltpu.VMEM((PK,), I32),
            pltpu.VMEM((PK + 16,), I32),
            pltpu.VMEM((PK + 16,), I32),
            pltpu.VMEM((SBP, 128), F32),
            pltpu.VMEM((CPT, 128), F32),
            pltpu.VMEM((CPT, 16), F32),
            pltpu.VMEM((CPT, 128), F32),
            pltpu.SemaphoreType.DMA,
        ],
    )


def _sc_pool_body(h_hbm, pkeys, sum_out, cnt_out, max_out,
                  key_v, qrow, qc, bufh, acc_s, acc_c, acc_m, sem):
    w = _wid()
    clo = w * CPT
    chi = clo + CPT

    z16 = jnp.zeros((16,), I32)

    def zeroq(i, c):
        sl = pl.ds(i * 16, 16)
        qrow[sl] = z16
        qc[sl] = z16
        return c

    lax.fori_loop(0, (PK + 16) // 16, zeroq, 0)

    zf = jnp.zeros((16,), F32)

    def zacc(i, c):
        acc_s[i >> 3, pl.ds((i & 7) * 16, 16)] = zf
        acc_m[i >> 3, pl.ds((i & 7) * 16, 16)] = zf
        return c

    lax.fori_loop(0, CPT * 8, zacc, 0)

    def zcnt(i, c):
        acc_c[i, pl.ds(0, 16)] = zf
        return c

    lax.fori_loop(0, CPT, zcnt, 0)

    ones = jnp.full((16,), 1.0, F32)

    def chunkf(ci, carry):
        pltpu.sync_copy(pkeys.at[pl.ds(ci * PK, PK)], key_v)

        def scan(v, qn):
            sl = pl.ds(v * 16, 16)
            kv = key_v[sl]
            cv = lax.shift_right_logical(kv, 16)
            m = (cv >= clo) & (cv < chi)
            plsc.store_compressed(qrow.at[pl.ds(qn, 16)], kv & 0xFFFF, mask=m)
            plsc.store_compressed(qc.at[pl.ds(qn, 16)], cv - clo, mask=m)
            return qn + jnp.sum(m.astype(I32))

        qn = lax.fori_loop(0, PK // 16, scan, 0)
        nb = (qn + SBP - 1) // SBP

        def dbody(b, c):
            cph = pltpu.async_copy(h_hbm.at[qrow.at[pl.ds(b * SBP, SBP)]],
                                   bufh, sem)
            cph.wait()
            nj = jnp.minimum(SBP, qn - b * SBP)

            def rmw(j, cc):
                ci_ = qc[pl.ds(b * SBP + j, 16)][0]
                for kk in range(8):
                    sl = pl.ds(kk * 16, 16)
                    v = bufh[j, sl]
                    acc_s[ci_, sl] = acc_s[ci_, sl] + v
                    acc_m[ci_, sl] = jnp.maximum(acc_m[ci_, sl], v)
                acc_c[ci_, pl.ds(0, 16)] = acc_c[ci_, pl.ds(0, 16)] + ones
                return cc

            lax.fori_loop(0, nj, rmw, 0)
            return c

        lax.fori_loop(0, nb, dbody, 0)
        return carry

    lax.fori_loop(0, POOL_CHUNKS, chunkf, 0)

    pltpu.sync_copy(acc_s, sum_out.at[pl.ds(clo, CPT)])
    pltpu.sync_copy(acc_c, cnt_out.at[pl.ds(clo, CPT)])
    pltpu.sync_copy(acc_m, max_out.at[pl.ds(clo, CPT)])


def kernel(x, edge_index, edge_attr, community, multi_community_nodes,
           multi_community_index, adj_inter, adj_intra, edge_attr_inter,
           edge_attr_intra, params):
    p = params

    def b2d(name):
        return p[name + "_b"].reshape(1, -1)

    pad_e = E_PAD - E
    zpad = jnp.zeros((pad_e,), I32)
    src_i = jnp.concatenate([adj_inter[0].astype(I32), zpad])
    dst_i = jnp.concatenate([adj_inter[1].astype(I32), zpad])
    src_a = jnp.concatenate([adj_intra[0].astype(I32), zpad])
    dst_a = jnp.concatenate([adj_intra[1].astype(I32), zpad])

    kn, km = _pack_pool_keys_call(
        community.astype(I32).reshape(25, 2000),
        multi_community_index.astype(I32).reshape(5, 2000),
        multi_community_nodes.astype(I32).reshape(5, 2000))
    pool_keys = jnp.concatenate([
        kn.reshape(N_NODES), km.reshape(N_MULTI),
        jnp.full((NPOOL - N_NODES - N_MULTI,), POOL_SENT, I32)])

    h0 = _embed_call(x, p["emb1_w"], b2d("emb1"), p["emb2_w"], b2d("emb2"),
                     p["emb3_w"], b2d("emb3"))

    zb = jnp.zeros((1, 128), F32)
    (a1, b1, a2, b2, mi1, ri1, ma1, ra1) = _node_mats_call(
        h0,
        [(p["el1_n_w"][:128], b2d("el1_n")), (p["el1_n_w"][128:], zb),
         (p["el2_n_w"][:128], b2d("el2_n")), (p["el2_n_w"][128:], zb),
         (p["ci1_m_w"], b2d("ci1_m")), (p["ci1_r_w"], b2d("ci1_r")),
         (p["ca1_m_w"], b2d("ca1_m")), (p["ca1_r_w"], b2d("ca1_r"))],
    )

    ga = _sc_gather_add()
    z0_i, keys_i = ga(a1, b1, src_i, dst_i)
    z0_a, keys_a = ga(a2, b2, src_a, dst_a)

    g1i, g2i = _edge_g_call(
        z0_i, edge_attr_inter, p["ee1_w"], b2d("ee1"),
        p["el1_e_w"], b2d("el1_e"),
        p["ci1_g_w"], b2d("ci1_g"), p["ci2_g_w"], b2d("ci2_g"))
    g1a, g2a = _edge_g_call(
        z0_a, edge_attr_intra, p["ee2_w"], b2d("ee2"),
        p["el2_e_w"], b2d("el2_e"),
        p["ca1_g_w"], b2d("ca1_g"), p["ca2_g_w"], b2d("ca2_g"))

    conv = _sc_conv()
    agg_i1 = conv(mi1, g1i, keys_i)
    agg_a1 = conv(ma1, g1a, keys_a)

    (h1, mi2, ri2, ma2, ra2) = _combine_call(
        ri1, agg_i1, ra1, agg_a1,
        [(p["ci2_m_w"], b2d("ci2_m")), (p["ci2_r_w"], b2d("ci2_r")),
         (p["ca2_m_w"], b2d("ca2_m")), (p["ca2_r_w"], b2d("ca2_r"))],
    )

    s1, c1, m1 = _sc_pool()(h1, pool_keys)

    agg_i2 = conv(mi2, g2i, keys_i)
    agg_a2 = conv(ma2, g2a, keys_a)

    (h2,) = _combine_call(ri2, agg_i2, ra2, agg_a2, [])

    s2, c2, m2 = _sc_pool()(h2, pool_keys)

    out = _final_call(s1, c1, m1, s2, c2, m2,
                      p["lin1_w"], b2d("lin1"), p["lin2_w"], b2d("lin2"))
    return out.reshape(N_COMM)


# speedup vs baseline: 6.9414x; 1.4275x over previous
import functools

import jax
import jax.numpy as jnp
from jax import lax
from jax.experimental import pallas as pl
from jax.experimental.pallas import tpu as pltpu
from jax.experimental.pallas import tpu_sc as plsc

F32 = jnp.float32
I32 = jnp.int32

N_NODES = 50000
N_COMM = 5000
E = 400000
N_MULTI = 10000

NW = 32
E_PAD = 401408
EPT = E_PAD // NW
GA_C = 256
GA_CHUNKS = EPT // GA_C

NPS = 6272
N_NODES_ACC = 8 * NPS
SPS = 4
RWT = NPS // 16
EPS = E_PAD // 16
CK = 1568
CONV_CHUNKS = EPS // CK
SB = 64
ZR = 56
CONV_SENT = 60000 << 16

CPT = 160
N_COMM_ACC = NW * CPT
NPOOL = 60416
PK = 1888
POOL_CHUNKS = NPOOL // PK
POOL_SENT = 8192 << 16
SBP = 64

NODE_BLK = 2000
EDGE_BLK = 2000
COMM_BLK = 1000

@functools.lru_cache(maxsize=None)
def _sc_mesh():
    return plsc.VectorSubcoreMesh(core_axis_name="c", subcore_axis_name="s")


def _wid():
    return lax.axis_index("s") * 2 + lax.axis_index("c")


def _full(shape):
    nd = len(shape)
    return pl.BlockSpec(shape, lambda i: (0,) * nd)


def _embed_call(x, w1, b1, w2, b2, w3, b3):
    def body(x_ref, w1r, b1r, w2r, b2r, w3r, b3r, h_ref):
        xb = x_ref[...]
        x1 = jax.nn.relu(xb[:, :8] @ w1r[...] + b1r[...])
        x2 = jax.nn.relu(xb[:, 8:] @ w2r[...] + b2r[...])
        hh = jnp.concatenate([x1, x2], axis=1)
        h_ref[...] = jax.nn.relu(hh @ w3r[...] + b3r[...])

    return pl.pallas_call(
        body,
        grid=(N_NODES // NODE_BLK,),
        in_specs=[
            pl.BlockSpec((NODE_BLK, 20), lambda i: (i, 0)),
            _full((8, 64)), _full((1, 64)),
            _full((12, 64)), _full((1, 64)),
            _full((128, 128)), _full((1, 128)),
        ],
        out_specs=pl.BlockSpec((NODE_BLK, 128), lambda i: (i, 0)),
        out_shape=jax.ShapeDtypeStruct((N_NODES, 128), F32),
    )(x, w1, b1, w2, b2, w3, b3)


def _node_mats_call(h, wbs):
    k = len(wbs)

    def body(*refs):
        h_ref = refs[0]
        w_refs = refs[1:1 + k]
        b_refs = refs[1 + k:1 + 2 * k]
        out_refs = refs[1 + 2 * k:]
        hb = h_ref[...]
        for j in range(k):
            out_refs[j][...] = hb @ w_refs[j][...] + b_refs[j][...]

    out_specs = [pl.BlockSpec((NODE_BLK, 128), lambda i: (i, 0))] * k
    out_shape = [jax.ShapeDtypeStruct((N_NODES, 128), F32)] * k
    args = [h] + [w for (w, _) in wbs] + [b for (_, b) in wbs]
    return pl.pallas_call(
        body,
        grid=(N_NODES // NODE_BLK,),
        in_specs=[pl.BlockSpec((NODE_BLK, 128), lambda i: (i, 0))]
        + [_full((128, 128))] * k + [_full((1, 128))] * k,
        out_specs=out_specs,
        out_shape=out_shape,
    )(*args)


def _edge_g_call(z0, ea, wee, bee, wel, bel, wg1, bg1, wg2, bg2):
    def body(z_ref, ea_ref, weer, beer, welr, belr, wg1r, bg1r, wg2r, bg2r,
             o1, o2):
        e2 = jax.nn.relu(ea_ref[...] @ weer[...] + beer[...])
        e1 = e2 @ welr[...] + belr[...]
        mask = jnp.tanh(z_ref[...] + e1)
        o1[...] = jax.nn.sigmoid(mask @ wg1r[...] + bg1r[...])
        o2[...] = jax.nn.sigmoid(mask @ wg2r[...] + bg2r[...])

    eo = [pl.BlockSpec((EDGE_BLK, 128), lambda i: (i, 0))] * 2
    es = [jax.ShapeDtypeStruct((E, 128), F32)] * 2
    return pl.pallas_call(
        body,
        grid=(E // EDGE_BLK,),
        in_specs=[
            pl.BlockSpec((EDGE_BLK, 128), lambda i: (i, 0)),
            pl.BlockSpec((EDGE_BLK, 16), lambda i: (i, 0)),
            _full((16, 64)), _full((1, 64)),
            _full((64, 128)), _full((1, 128)),
            _full((128, 128)), _full((1, 128)),
            _full((128, 128)), _full((1, 128)),
        ],
        out_specs=eo,
        out_shape=es,
    )(z0, ea, wee, bee, wel, bel, wg1, bg1, wg2, bg2)


def _combine_call(ri, ai, ra, aa, wbs):
    k = len(wbs)

    def body(*refs):
        rir, air, rar, aar = refs[:4]
        w_refs = refs[4:4 + k]
        b_refs = refs[4 + k:4 + 2 * k]
        outs = refs[4 + 2 * k:]
        hb = jax.nn.relu(rir[...] + air[...]) + jax.nn.relu(rar[...] + aar[...])
        outs[0][...] = hb
        for j in range(k):
            outs[1 + j][...] = hb @ w_refs[j][...] + b_refs[j][...]

    out_specs = [pl.BlockSpec((NODE_BLK, 128), lambda i: (i, 0))] * (1 + k)
    out_shape = [jax.ShapeDtypeStruct((N_NODES, 128), F32)] * (1 + k)

    args = [ri, ai, ra, aa] + [w for (w, _) in wbs] + [b for (_, b) in wbs]
    return pl.pallas_call(
        body,
        grid=(N_NODES // NODE_BLK,),
        in_specs=[pl.BlockSpec((NODE_BLK, 128), lambda i: (i, 0))] * 4
        + [_full((128, 128))] * k + [_full((1, 128))] * k,
        out_specs=out_specs,
        out_shape=out_shape,
    )(*args)


def _final_call(s1, c1, m1, s2, c2, m2, wl1, bl1, wl2, bl2):
    def body(s1r, c1r, m1r, s2r, c2r, m2r, w1r, b1r, w2r, b2r, out_ref):
        mean1 = s1r[...] / jnp.maximum(c1r[...][:, :1], 1.0)
        mean2 = s2r[...] / jnp.maximum(c2r[...][:, :1], 1.0)
        g = (jnp.concatenate([mean1, m1r[...]], axis=1)
             + jnp.concatenate([mean2, m2r[...]], axis=1))
        gg = jax.nn.relu(g @ w1r[...] + b1r[...])
        out_ref[...] = gg @ w2r[...] + b2r[...]

    return pl.pallas_call(
        body,
        grid=(N_COMM // COMM_BLK,),
        in_specs=[
            pl.BlockSpec((COMM_BLK, 128), lambda i: (i, 0)),
            pl.BlockSpec((COMM_BLK, 16), lambda i: (i, 0)),
            pl.BlockSpec((COMM_BLK, 128), lambda i: (i, 0)),
            pl.BlockSpec((COMM_BLK, 128), lambda i: (i, 0)),
            pl.BlockSpec((COMM_BLK, 16), lambda i: (i, 0)),
            pl.BlockSpec((COMM_BLK, 128), lambda i: (i, 0)),
            _full((256, 128)), _full((1, 128)),
            _full((128, 1)), _full((1, 1)),
        ],
        out_specs=pl.BlockSpec((COMM_BLK, 1), lambda i: (i, 0)),
        out_shape=jax.ShapeDtypeStruct((N_COMM, 1), F32),
    )(s1, c1, m1, s2, c2, m2, wl1, bl1, wl2, bl2)


def _pack_pool_keys_call(comm2d, mci2d, mcn2d):
    def nbody(c_ref, o_ref):
        r = jax.lax.broadcasted_iota(I32, (25, 2000), 0)
        cc = jax.lax.broadcasted_iota(I32, (25, 2000), 1)
        o_ref[...] = (c_ref[...] << 16) | (r * 2000 + cc)

    def mbody(ci_ref, cn_ref, o_ref):
        o_ref[...] = (ci_ref[...] << 16) | cn_ref[...]

    kn = pl.pallas_call(
        nbody,
        grid=(1,),
        in_specs=[pl.BlockSpec((25, 2000), lambda i: (0, 0))],
        out_specs=pl.BlockSpec((25, 2000), lambda i: (0, 0)),
        out_shape=jax.ShapeDtypeStruct((25, 2000), I32),
    )(comm2d)
    km = pl.pallas_call(
        mbody,
        grid=(1,),
        in_specs=[pl.BlockSpec((5, 2000), lambda i: (0, 0))] * 2,
        out_specs=pl.BlockSpec((5, 2000), lambda i: (0, 0)),
        out_shape=jax.ShapeDtypeStruct((5, 2000), I32),
    )(mci2d, mcn2d)
    return kn, km


@functools.lru_cache(maxsize=None)
def _sc_gather_add():
    return pl.kernel(
        _sc_gather_add_body,
        mesh=_sc_mesh(),
        compiler_params=pltpu.CompilerParams(needs_layout_passes=False),
        out_type=(
            jax.ShapeDtypeStruct((E_PAD, 128), F32),
            jax.ShapeDtypeStruct((E_PAD,), I32),
        ),
        scratch_types=[
            pltpu.VMEM((GA_C,), I32),
            pltpu.VMEM((GA_C,), I32),
            pltpu.VMEM((GA_C, 128), F32),
            pltpu.VMEM((GA_C, 128), F32),
            pltpu.VMEM((GA_C,), I32),
            pltpu.SemaphoreType.DMA,
            pltpu.SemaphoreType.DMA,
        ],
    )


def _sc_gather_add_body(a_hbm, b_hbm, src_hbm, dst_hbm, z_out, key_out,
                   src_v, dst_v, bufa, bufb, key_v, sem1, sem2):
    base0 = _wid() * EPT

    def chunk(ci, carry):
        base = base0 + ci * GA_C
        pltpu.sync_copy(src_hbm.at[pl.ds(base, GA_C)], src_v)
        pltpu.sync_copy(dst_hbm.at[pl.ds(base, GA_C)], dst_v)
        cpa = pltpu.async_copy(a_hbm.at[src_v], bufa, sem1)
        cpb = pltpu.async_copy(b_hbm.at[dst_v], bufb, sem2)
        cpa.wait()
        cpb.wait()

        def addrow(r, c):
            for kk in range(8):
                sl = pl.ds(kk * 16, 16)
                bufa[r, sl] = bufa[r, sl] + bufb[r, sl]
            return c

        lax.fori_loop(0, GA_C, addrow, 0)

        iota16 = lax.iota(I32, 16)

        def keyvec(v, c):
            sl = pl.ds(v * 16, 16)
            sv = src_v[sl]
            dv = dst_v[sl]
            gi = base + v * 16 + iota16
            key_v[sl] = jnp.where(gi < E, (dv << 16) | sv,
                                  jnp.full((16,), CONV_SENT, I32))
            return c

        lax.fori_loop(0, GA_C // 16, keyvec, 0)
        pltpu.sync_copy(bufa, z_out.at[pl.ds(base, GA_C)])
        pltpu.sync_copy(key_v, key_out.at[pl.ds(base, GA_C)])
        return carry

    lax.fori_loop(0, GA_CHUNKS, chunk, 0)


@functools.lru_cache(maxsize=None)
def _sc_conv():
    return pl.kernel(
        _sc_conv_body,
        mesh=_sc_mesh(),
        compiler_params=pltpu.CompilerParams(needs_layout_passes=False),
        out_type=jax.ShapeDtypeStruct((N_NODES_ACC, 128), F32),
        scratch_types=[
            pltpu.VMEM((CK,), I32),
            pltpu.VMEM((CK + 16 + SB,), I32),
            pltpu.VMEM((CK + 16 + SB,), I32),
            pltpu.VMEM((CK + 16 + SB,), I32),
            pltpu.VMEM((SB, 128), F32),
            pltpu.VMEM((SB, 128), F32),
            pltpu.VMEM((SB, 128), F32),
            pltpu.VMEM((ZR, 128), F32),
            pltpu.VMEM_SHARED((NPS + 16, 128), F32),
            pltpu.SemaphoreType.DMA,
            pltpu.SemaphoreType.DMA,
        ],
    )


def _sc_conv_body(hm, gt, keys, agg_out,
                  key_v, qsrc, qdst, qeid, bufh, bufg, msg, zbuf,
                  shared, sem1, sem2):
    c = lax.axis_index("c")
    sct = lax.axis_index("s")
    dummy = NPS + sct

    z16i = jnp.zeros((16,), I32)

    def zeroq(i, cc):
        sl = pl.ds(i * 16, 16)
        qsrc[sl] = z16i
        qdst[sl] = z16i
        qeid[sl] = z16i
        return cc

    lax.fori_loop(0, (CK + 16 + SB) // 16, zeroq, 0)

    zf = jnp.zeros((16,), F32)

    def zzb(i, cc):
        zbuf[i >> 3, pl.ds((i & 7) * 16, 16)] = zf
        return cc

    lax.fori_loop(0, ZR * 8, zzb, 0)

    iota16 = lax.iota(I32, 16)
    dumv = jnp.zeros((16,), I32) + dummy

    def qpass(q, carry):
        section = c * SPS + q
        lo = section * NPS
        hi = lo + NPS

        def clr(k, cc):
            pltpu.sync_copy(zbuf, shared.at[pl.ds(sct * RWT + k * ZR, ZR)])
            return cc

        lax.fori_loop(0, RWT // ZR, clr, 0)
        pltpu.sync_copy(zbuf.at[pl.ds(0, 16)], shared.at[pl.ds(NPS, 16)])
        plsc.subcore_barrier()

        def chunk(ci, carry2):
            base = sct * EPS + ci * CK
            pltpu.sync_copy(keys.at[pl.ds(base, CK)], key_v)

            def scan(v, qn):
                sl = pl.ds(v * 16, 16)
                kv = key_v[sl]
                dv = lax.shift_right_logical(kv, 16)
                m = (dv >= lo) & (dv < hi)
                plsc.store_compressed(qsrc.at[pl.ds(qn, 16)], kv & 0xFFFF,
                                      mask=m)
                plsc.store_compressed(qdst.at[pl.ds(qn, 16)], dv - lo, mask=m)
                plsc.store_compressed(qeid.at[pl.ds(qn, 16)],
                                      base + v * 16 + iota16, mask=m)
                return qn + jnp.sum(m.astype(I32))

            qn = lax.fori_loop(0, CK // 16, scan, 0)
            for t in range(SB // 16):
                qdst[pl.ds(qn + t * 16, 16)] = dumv
            nb = (qn + SB - 1) // SB

            def drain(b, cc):
                cph = pltpu.async_copy(hm.at[qsrc.at[pl.ds(b * SB, SB)]],
                                       bufh, sem1)
                cpg = pltpu.async_copy(gt.at[qeid.at[pl.ds(b * SB, SB)]],
                                       bufg, sem2)
                cph.wait()
                cpg.wait()

                def mul(r, c3):
                    for kk in range(8):
                        slm = pl.ds(kk * 16, 16)
                        msg[r, slm] = bufh[r, slm] * bufg[r, slm]
                    return c3

                lax.fori_loop(0, SB, mul, 0)
                pltpu.sync_copy(msg, shared.at[qdst.at[pl.ds(b * SB, SB)]],
                                add=True)
                return cc

            lax.fori_loop(0, nb, drain, 0)
            return carry2

        lax.fori_loop(0, CONV_CHUNKS, chunk, 0)
        plsc.subcore_barrier()
        pltpu.sync_copy(shared.at[pl.ds(sct * RWT, RWT)],
                        agg_out.at[pl.ds(lo + sct * RWT, RWT)])
        plsc.subcore_barrier()
        return carry

    lax.fori_loop(0, SPS, qpass, 0)


@functools.lru_cache(maxsize=None)
def _sc_pool():
    return pl.kernel(
        _sc_pool_body,
        mesh=_sc_mesh(),
        compiler_params=pltpu.CompilerParams(needs_layout_passes=False),
        out_type=(
            jax.ShapeDtypeStruct((N_COMM_ACC, 128), F32),
            jax.ShapeDtypeStruct((N_COMM_ACC, 16), F32),
            jax.ShapeDtypeStruct((N_COMM_ACC, 128), F32),
        ),
        scratch_types=[
            pltpu.VMEM((PK,), I32),
            pltpu.VMEM((PK + 16,), I32),
            pltpu.VMEM((PK + 16,), I32),
            pltpu.VMEM((SBP, 128), F32),
            pltpu.VMEM((CPT, 128), F32),
            pltpu.VMEM((CPT, 16), F32),
            pltpu.VMEM((CPT, 128), F32),
            pltpu.SemaphoreType.DMA,
        ],
    )


def _sc_pool_body(h_hbm, pkeys, sum_out, cnt_out, max_out,
                  key_v, qrow, qc, bufh, acc_s, acc_c, acc_m, sem):
    w = _wid()
    clo = w * CPT
    chi = clo + CPT

    z16 = jnp.zeros((16,), I32)

    def zeroq(i, c):
        sl = pl.ds(i * 16, 16)
        qrow[sl] = z16
        qc[sl] = z16
        return c

    lax.fori_loop(0, (PK + 16) // 16, zeroq, 0)

    zf = jnp.zeros((16,), F32)

    def zacc(i, c):
        acc_s[i >> 3, pl.ds((i & 7) * 16, 16)] = zf
        acc_m[i >> 3, pl.ds((i & 7) * 16, 16)] = zf
        return c

    lax.fori_loop(0, CPT * 8, zacc, 0)

    def zcnt(i, c):
        acc_c[i, pl.ds(0, 16)] = zf
        return c

    lax.fori_loop(0, CPT, zcnt, 0)

    ones = jnp.full((16,), 1.0, F32)

    def chunkf(ci, carry):
        pltpu.sync_copy(pkeys.at[pl.ds(ci * PK, PK)], key_v)

        def scan(v, qn):
            sl = pl.ds(v * 16, 16)
            kv = key_v[sl]
            cv = lax.shift_right_logical(kv, 16)
            m = (cv >= clo) & (cv < chi)
            plsc.store_compressed(qrow.at[pl.ds(qn, 16)], kv & 0xFFFF, mask=m)
            plsc.store_compressed(qc.at[pl.ds(qn, 16)], cv - clo, mask=m)
            return qn + jnp.sum(m.astype(I32))

        qn = lax.fori_loop(0, PK // 16, scan, 0)
        nb = (qn + SBP - 1) // SBP

        def dbody(b, c):
            cph = pltpu.async_copy(h_hbm.at[qrow.at[pl.ds(b * SBP, SBP)]],
                                   bufh, sem)
            cph.wait()
            nj = jnp.minimum(SBP, qn - b * SBP)

            def rmw(j, cc):
                ci_ = qc[pl.ds(b * SBP + j, 16)][0]
                for kk in range(8):
                    sl = pl.ds(kk * 16, 16)
                    v = bufh[j, sl]
                    acc_s[ci_, sl] = acc_s[ci_, sl] + v
                    acc_m[ci_, sl] = jnp.maximum(acc_m[ci_, sl], v)
                acc_c[ci_, pl.ds(0, 16)] = acc_c[ci_, pl.ds(0, 16)] + ones
                return cc

            lax.fori_loop(0, nj, rmw, 0)
            return c

        lax.fori_loop(0, nb, dbody, 0)
        return carry

    lax.fori_loop(0, POOL_CHUNKS, chunkf, 0)

    pltpu.sync_copy(acc_s, sum_out.at[pl.ds(clo, CPT)])
    pltpu.sync_copy(acc_c, cnt_out.at[pl.ds(clo, CPT)])
    pltpu.sync_copy(acc_m, max_out.at[pl.ds(clo, CPT)])


def kernel(x, edge_index, edge_attr, community, multi_community_nodes,
           multi_community_index, adj_inter, adj_intra, edge_attr_inter,
           edge_attr_intra, params):
    p = params

    def b2d(name):
        return p[name + "_b"].reshape(1, -1)

    pad_e = E_PAD - E
    zpad = jnp.zeros((pad_e,), I32)
    src_i = jnp.concatenate([adj_inter[0].astype(I32), zpad])
    dst_i = jnp.concatenate([adj_inter[1].astype(I32), zpad])
    src_a = jnp.concatenate([adj_intra[0].astype(I32), zpad])
    dst_a = jnp.concatenate([adj_intra[1].astype(I32), zpad])

    kn, km = _pack_pool_keys_call(
        community.astype(I32).reshape(25, 2000),
        multi_community_index.astype(I32).reshape(5, 2000),
        multi_community_nodes.astype(I32).reshape(5, 2000))
    pool_keys = jnp.concatenate([
        kn.reshape(N_NODES), km.reshape(N_MULTI),
        jnp.full((NPOOL - N_NODES - N_MULTI,), POOL_SENT, I32)])

    h0 = _embed_call(x, p["emb1_w"], b2d("emb1"), p["emb2_w"], b2d("emb2"),
                     p["emb3_w"], b2d("emb3"))

    zb = jnp.zeros((1, 128), F32)
    (a1, b1, a2, b2, mi1, ri1, ma1, ra1) = _node_mats_call(
        h0,
        [(p["el1_n_w"][:128], b2d("el1_n")), (p["el1_n_w"][128:], zb),
         (p["el2_n_w"][:128], b2d("el2_n")), (p["el2_n_w"][128:], zb),
         (p["ci1_m_w"], b2d("ci1_m")), (p["ci1_r_w"], b2d("ci1_r")),
         (p["ca1_m_w"], b2d("ca1_m")), (p["ca1_r_w"], b2d("ca1_r"))],
    )

    ga = _sc_gather_add()
    z0_i, keys_i = ga(a1, b1, src_i, dst_i)
    z0_a, keys_a = ga(a2, b2, src_a, dst_a)

    g1i, g2i = _edge_g_call(
        z0_i, edge_attr_inter, p["ee1_w"], b2d("ee1"),
        p["el1_e_w"], b2d("el1_e"),
        p["ci1_g_w"], b2d("ci1_g"), p["ci2_g_w"], b2d("ci2_g"))
    g1a, g2a = _edge_g_call(
        z0_a, edge_attr_intra, p["ee2_w"], b2d("ee2"),
        p["el2_e_w"], b2d("el2_e"),
        p["ca1_g_w"], b2d("ca1_g"), p["ca2_g_w"], b2d("ca2_g"))

    conv = _sc_conv()
    agg_i1 = conv(mi1, g1i, keys_i)
    agg_a1 = conv(ma1, g1a, keys_a)

    (h1, mi2, ri2, ma2, ra2) = _combine_call(
        ri1, agg_i1, ra1, agg_a1,
        [(p["ci2_m_w"], b2d("ci2_m")), (p["ci2_r_w"], b2d("ci2_r")),
         (p["ca2_m_w"], b2d("ca2_m")), (p["ca2_r_w"], b2d("ca2_r"))],
    )

    s1, c1, m1 = _sc_pool()(h1, pool_keys)

    agg_i2 = conv(mi2, g2i, keys_i)
    agg_a2 = conv(ma2, g2a, keys_a)

    (h2,) = _combine_call(ri2, agg_i2, ra2, agg_a2, [])

    s2, c2, m2 = _sc_pool()(h2, pool_keys)

    out = _final_call(s1, c1, m1, s2, c2, m2,
                      p["lin1_w"], b2d("lin1"), p["lin2_w"], b2d("lin2"))
    return out.reshape(N_COMM)
